# baseline (speedup 1.0000x reference)
import functools

import jax
import jax.numpy as jnp
from jax import lax
from jax.experimental import pallas as pl
from jax.experimental.pallas import tpu as pltpu
from jax.experimental.pallas import tpu_sc as plsc

_C = 100
_D = 256
_DH = 128
_B = 4096
_BSC = 1536
_TCBLK = 512
_NTCB = (_B - _BSC) // _TCBLK
_TLAMBDA = 0.01

_NC = 2
_NS = 16
_NW = _NC * _NS
_BPW = _BSC // _NW
_LANES = 16
_NCHUNK = 3
_CH = _BPW // _NCHUNK

_sc_mesh = plsc.VectorSubcoreMesh(core_axis_name="c", subcore_axis_name="s")


@functools.partial(
    pl.kernel,
    mesh=_sc_mesh,
    out_type=jax.ShapeDtypeStruct((_NW, _LANES), jnp.float32),
    scratch_types=[
        pltpu.VMEM((_BPW,), jnp.int32),
        pltpu.VMEM((_BPW,), jnp.int32),
        pltpu.VMEM((_BPW,), jnp.int32),
        pltpu.VMEM((_BPW, _DH), jnp.float32),
        pltpu.VMEM((_BPW, _DH), jnp.float32),
        pltpu.VMEM((_BPW, _D), jnp.float32),
        pltpu.VMEM((_LANES,), jnp.float32),
        pltpu.SemaphoreType.DMA,
        pltpu.SemaphoreType.DMA,
        pltpu.SemaphoreType.DMA,
    ],
)
def _sc_kernel(label_hbm, feat_hbm, tab_hbm, out_hbm,
               idx_v, idxa_v, idxb_v, gath_a, gath_b, feat_v, out_v,
               semf, sema, semb):
    wid = lax.axis_index("s") * _NC + lax.axis_index("c")
    base = wid * _BPW
    pltpu.sync_copy(label_hbm.at[pl.ds(base, _BPW)], idx_v)

    def mkidx(j, _):
        two = idx_v[pl.ds(j * _LANES, _LANES)] * 2
        idxa_v[pl.ds(j * _LANES, _LANES)] = two
        idxb_v[pl.ds(j * _LANES, _LANES)] = two + 1
        return 0
    lax.fori_loop(0, _BPW // _LANES, mkidx, 0)

    copies = []
    for c in range(_NCHUNK):
        s = c * _CH
        copies.append((
            pltpu.async_copy(feat_hbm.at[pl.ds(base + s, _CH)],
                             feat_v.at[pl.ds(s, _CH)], semf),
            pltpu.async_copy(tab_hbm.at[idxa_v.at[pl.ds(s, _CH)]],
                             gath_a.at[pl.ds(s, _CH)], sema),
            pltpu.async_copy(tab_hbm.at[idxb_v.at[pl.ds(s, _CH)]],
                             gath_b.at[pl.ds(s, _CH)], semb),
        ))

    def row_body(i, acc):
        a = acc
        for k in range(_DH // _LANES):
            dlt = (feat_v[i, pl.ds(k * _LANES, _LANES)]
                   - gath_a[i, pl.ds(k * _LANES, _LANES)])
            a = a + dlt * dlt
        for k in range(_DH // _LANES):
            dlt = (feat_v[i, pl.ds(_DH + k * _LANES, _LANES)]
                   - gath_b[i, pl.ds(k * _LANES, _LANES)])
            a = a + dlt * dlt
        return a

    acc_c = jnp.zeros((_LANES,), jnp.float32)
    for c in range(_NCHUNK):
        for cp in copies[c]:
            cp.wait()
        acc_c = lax.fori_loop(c * _CH, (c + 1) * _CH, row_body, acc_c)

    out_v[...] = acc_c
    pltpu.sync_copy(out_v, out_hbm.at[wid])


def _tc_kernel(cen_ref, lab_ref, feat_ref, out_ref):
    i = pl.program_id(0)
    cen = cen_ref[...]
    labc = lab_ref[...]

    @pl.when(i == 0)
    def _first():
        g = lax.dot_general(cen, cen, (((1,), (1,)), ((), ())),
                            preferred_element_type=jnp.float32)
        norm2 = jnp.sum(cen * cen, axis=1)
        n = jnp.sqrt(norm2)
        n_safe = jnp.where(n > 0.0, n, 1.0)
        cos = g / (n_safe[:, None] * n_safe[None, :])
        row = lax.broadcasted_iota(jnp.int32, (_C, _C), 0)
        col = lax.broadcasted_iota(jnp.int32, (_C, _C), 1)
        diag = jnp.sum(jnp.where(row == col, cos, 0.0), axis=1)
        r = jnp.sum(cos, axis=1) - diag + (_C - 1.0)
        cls = lax.broadcasted_iota(jnp.int32, (_B, _C), 1)
        ohall = (labc == cls).astype(jnp.float32)
        count = jnp.sum(ohall, axis=0)
        island = jnp.sum(count * r)
        out_ref[...] = jnp.full((1, 1), _TLAMBDA * island, jnp.float32)

    lab_blk = lab_ref[pl.ds(_BSC + i * _TCBLK, _TCBLK), :]
    cls_b = lax.broadcasted_iota(jnp.int32, (_TCBLK, _C), 1)
    oh = (lab_blk == cls_b).astype(jnp.float32)
    cb = lax.dot_general(oh, cen, (((1,), (0,)), ((), ())),
                         preferred_element_type=jnp.float32)
    fh = feat_ref[...]
    dlt = fh - cb
    center_blk = jnp.sum(dlt * dlt)
    out_ref[...] = out_ref[...] + center_blk * (0.5 / _B)


def kernel(label, feat, centers, tlabel):
    del tlabel
    label = label.astype(jnp.int32)
    feat = feat.astype(jnp.float32)
    centers = centers.astype(jnp.float32)
    tab = centers.reshape(2 * _C, _DH)
    parts = _sc_kernel(label, feat, tab)
    tc_total = pl.pallas_call(
        _tc_kernel,
        grid=(_NTCB,),
        in_specs=[
            pl.BlockSpec((_C, _D), lambda i: (0, 0)),
            pl.BlockSpec((_B, 1), lambda i: (0, 0)),
            pl.BlockSpec((_TCBLK, _D), lambda i: (_BSC // _TCBLK + i, 0)),
        ],
        out_specs=pl.BlockSpec((1, 1), lambda i: (0, 0)),
        out_shape=jax.ShapeDtypeStruct((1, 1), jnp.float32),
    )(centers, label[:, None], feat)
    return tc_total.reshape(()) + jnp.sum(parts) * (0.5 / _B)


# speedup vs baseline: 1.0222x; 1.0222x over previous
import functools

import jax
import jax.numpy as jnp
from jax import lax
from jax.experimental import pallas as pl
from jax.experimental.pallas import tpu as pltpu
from jax.experimental.pallas import tpu_sc as plsc

_C = 100
_D = 256
_DH = 128
_B = 4096
_BSC = 2048
_TLAMBDA = 0.01

_NC = 2
_NS = 16
_NW = _NC * _NS
_BPW = _BSC // _NW
_LANES = 16
_NCHUNK = 4
_CH = _BPW // _NCHUNK

_sc_mesh = plsc.VectorSubcoreMesh(core_axis_name="c", subcore_axis_name="s")


@functools.partial(
    pl.kernel,
    mesh=_sc_mesh,
    out_type=jax.ShapeDtypeStruct((_NW, _LANES), jnp.float32),
    scratch_types=[
        pltpu.VMEM((_BPW,), jnp.int32),
        pltpu.VMEM((_BPW,), jnp.int32),
        pltpu.VMEM((_BPW,), jnp.int32),
        pltpu.VMEM((_BPW, _DH), jnp.float32),
        pltpu.VMEM((_BPW, _DH), jnp.float32),
        pltpu.VMEM((_BPW, _D), jnp.float32),
        pltpu.VMEM((_LANES,), jnp.float32),
        pltpu.SemaphoreType.DMA,
        pltpu.SemaphoreType.DMA,
        pltpu.SemaphoreType.DMA,
    ],
)
def _sc_kernel(label_hbm, feat_hbm, tab_hbm, out_hbm,
               idx_v, idxa_v, idxb_v, gath_a, gath_b, feat_v, out_v,
               semf, sema, semb):
    wid = lax.axis_index("s") * _NC + lax.axis_index("c")
    base = wid * _BPW
    pltpu.sync_copy(label_hbm.at[pl.ds(base, _BPW)], idx_v)

    def mkidx(j, _):
        two = idx_v[pl.ds(j * _LANES, _LANES)] * 2
        idxa_v[pl.ds(j * _LANES, _LANES)] = two
        idxb_v[pl.ds(j * _LANES, _LANES)] = two + 1
        return 0
    lax.fori_loop(0, _BPW // _LANES, mkidx, 0)

    copies = []
    for c in range(_NCHUNK):
        s = c * _CH
        copies.append((
            pltpu.async_copy(feat_hbm.at[pl.ds(base + s, _CH)],
                             feat_v.at[pl.ds(s, _CH)], semf),
            pltpu.async_copy(tab_hbm.at[idxa_v.at[pl.ds(s, _CH)]],
                             gath_a.at[pl.ds(s, _CH)], sema),
            pltpu.async_copy(tab_hbm.at[idxb_v.at[pl.ds(s, _CH)]],
                             gath_b.at[pl.ds(s, _CH)], semb),
        ))

    def row_body(i, acc):
        a = acc
        for k in range(_DH // _LANES):
            dlt = (feat_v[i, pl.ds(k * _LANES, _LANES)]
                   - gath_a[i, pl.ds(k * _LANES, _LANES)])
            a = a + dlt * dlt
        for k in range(_DH // _LANES):
            dlt = (feat_v[i, pl.ds(_DH + k * _LANES, _LANES)]
                   - gath_b[i, pl.ds(k * _LANES, _LANES)])
            a = a + dlt * dlt
        return a

    acc_c = jnp.zeros((_LANES,), jnp.float32)
    for c in range(_NCHUNK):
        for cp in copies[c]:
            cp.wait()
        acc_c = lax.fori_loop(c * _CH, (c + 1) * _CH, row_body, acc_c)

    out_v[...] = acc_c
    pltpu.sync_copy(out_v, out_hbm.at[wid])


def _tc_kernel(cen_ref, lab_ref, feat_ref, out_ref):
    cen = cen_ref[...]
    g = lax.dot_general(cen, cen, (((1,), (1,)), ((), ())),
                        preferred_element_type=jnp.float32)
    norm2 = jnp.sum(cen * cen, axis=1)
    n = jnp.sqrt(norm2)
    n_safe = jnp.where(n > 0.0, n, 1.0)
    cos = g / (n_safe[:, None] * n_safe[None, :])
    row = lax.broadcasted_iota(jnp.int32, (_C, _C), 0)
    col = lax.broadcasted_iota(jnp.int32, (_C, _C), 1)
    diag = jnp.sum(jnp.where(row == col, cos, 0.0), axis=1)
    r = jnp.sum(cos, axis=1) - diag + (_C - 1.0)

    labc = lab_ref[...]
    cls = lax.broadcasted_iota(jnp.int32, (_B, _C), 1)
    ohall = (labc == cls).astype(jnp.float32)
    count = jnp.sum(ohall, axis=0)
    island = jnp.sum(count * r)

    oh_hi = ohall[_BSC:, :]
    cb = lax.dot_general(oh_hi, cen, (((1,), (0,)), ((), ())),
                         preferred_element_type=jnp.float32)
    fh = feat_ref[...]
    dlt = fh - cb
    center_hi = jnp.sum(dlt * dlt)

    total = _TLAMBDA * island + center_hi * (0.5 / _B)
    out_ref[...] = jnp.full((1, 1), total, jnp.float32)


def kernel(label, feat, centers, tlabel):
    del tlabel
    label = label.astype(jnp.int32)
    feat = feat.astype(jnp.float32)
    centers = centers.astype(jnp.float32)
    tab = centers.reshape(2 * _C, _DH)
    parts = _sc_kernel(label, feat, tab)
    tc_total = pl.pallas_call(
        _tc_kernel,
        grid=(1,),
        in_specs=[
            pl.BlockSpec((_C, _D), lambda i: (0, 0)),
            pl.BlockSpec((_B, 1), lambda i: (0, 0)),
            pl.BlockSpec((_BSC, _D), lambda i: (1, 0)),
        ],
        out_specs=pl.BlockSpec((1, 1), lambda i: (0, 0)),
        out_shape=jax.ShapeDtypeStruct((1, 1), jnp.float32),
    )(centers, label[:, None], feat)
    return tc_total.reshape(()) + jnp.sum(parts) * (0.5 / _B)
